# Initial kernel scaffold; baseline (speedup 1.0000x reference)
import jax
import jax.numpy as jnp
from jax.experimental import pallas as pl


def kernel(seqs, lens):
    raise NotImplementedError("write your pallas kernel here")


# speedup vs baseline: 4.4569x; 4.4569x over previous
import functools

import jax
import jax.numpy as jnp
from jax.experimental import pallas as pl

NSTACK = 4
BT = 128


def _body(lens_ref, halo_ref, cur_ref, out_ref):
    k = pl.program_id(0)
    cur = cur_ref[...]
    halo = halo_ref[...]
    bt, b, d = cur.shape
    t_abs = k * bt + jax.lax.broadcasted_iota(jnp.int32, (bt, b, d), 0)
    keep = t_abs < lens_ref[...]
    first = cur[0:1]
    parts = []
    for i in range(NSTACK):
        sh = NSTACK - 1 - i
        if sh == 0:
            part = cur
        else:
            tail = jnp.where(k == 0,
                             jnp.broadcast_to(first, (sh, b, d)),
                             halo[8 - sh:8])
            part = jnp.concatenate([tail, cur[:bt - sh]], axis=0)
        parts.append(jnp.where(keep, part, 0.0))
    out_ref[...] = jnp.concatenate(parts, axis=-1)


@jax.jit
def kernel(seqs, lens):
    T, B, D = seqs.shape
    grid = (T // BT,)
    lens3d = jnp.broadcast_to(lens.reshape(1, B, 1), (1, B, D))
    out = pl.pallas_call(
        _body,
        grid=grid,
        in_specs=[
            pl.BlockSpec((1, B, D), lambda k: (0, 0, 0)),
            pl.BlockSpec((8, B, D), lambda k: (jnp.maximum(k * (BT // 8) - 1, 0), 0, 0)),
            pl.BlockSpec((BT, B, D), lambda k: (k, 0, 0)),
        ],
        out_specs=pl.BlockSpec((BT, B, NSTACK * D), lambda k: (k, 0, 0)),
        out_shape=jax.ShapeDtypeStruct((T, B, NSTACK * D), seqs.dtype),
    )(lens3d, seqs, seqs)
    return (out, lens)


# speedup vs baseline: 4.6236x; 1.0374x over previous
import functools

import jax
import jax.numpy as jnp
from jax.experimental import pallas as pl

NSTACK = 4
BT = 256


def _body(lens_ref, halo_ref, cur_ref, out_ref):
    k = pl.program_id(0)
    cur = cur_ref[...]
    halo = halo_ref[...]
    bt, b, d = cur.shape
    t_abs = k * bt + jax.lax.broadcasted_iota(jnp.int32, (bt, b, d), 0)
    keep = t_abs < lens_ref[...]
    first = cur[0:1]
    parts = []
    for i in range(NSTACK):
        sh = NSTACK - 1 - i
        if sh == 0:
            part = cur
        else:
            tail = jnp.where(k == 0,
                             jnp.broadcast_to(first, (sh, b, d)),
                             halo[8 - sh:8])
            part = jnp.concatenate([tail, cur[:bt - sh]], axis=0)
        parts.append(jnp.where(keep, part, 0.0))
    out_ref[...] = jnp.concatenate(parts, axis=-1)


@jax.jit
def kernel(seqs, lens):
    T, B, D = seqs.shape
    grid = (T // BT,)
    lens3d = jnp.broadcast_to(lens.reshape(1, B, 1), (1, B, D))
    out = pl.pallas_call(
        _body,
        grid=grid,
        in_specs=[
            pl.BlockSpec((1, B, D), lambda k: (0, 0, 0)),
            pl.BlockSpec((8, B, D), lambda k: (jnp.maximum(k * (BT // 8) - 1, 0), 0, 0)),
            pl.BlockSpec((BT, B, D), lambda k: (k, 0, 0)),
        ],
        out_specs=pl.BlockSpec((BT, B, NSTACK * D), lambda k: (k, 0, 0)),
        out_shape=jax.ShapeDtypeStruct((T, B, NSTACK * D), seqs.dtype),
    )(lens3d, seqs, seqs)
    return (out, lens)
